# Initial kernel scaffold; baseline (speedup 1.0000x reference)
import jax
import jax.numpy as jnp
from jax.experimental import pallas as pl


def kernel(x, fixed_knn_graph, W1, g1, b1, W2, g2, b2):
    raise NotImplementedError("write your pallas kernel here")


# speedup vs baseline: 14.8824x; 14.8824x over previous
import functools

import jax
import jax.numpy as jnp
from jax import lax
from jax.experimental import pallas as pl
from jax.experimental.pallas import tpu as pltpu
from jax.experimental.pallas import tpu_sc as plsc

_NEG_SLOPE = 0.2
_EPS = 1e-5

_NC = 2
_NS = 16
_NW = _NC * _NS


def _lrelu(v):
    return jnp.where(v >= 0, v, _NEG_SLOPE * v)


def _proj_body(xt_ref, w_ref, pt_ref, qt_ref):
    rb = xt_ref.shape[0]
    y = jnp.dot(xt_ref[...], w_ref[...], preferred_element_type=jnp.float32)
    pt_ref[...] = jnp.concatenate(
        [y[:, :64], jnp.zeros((rb, 64), jnp.float32)], axis=1
    )
    qt_ref[...] = y[:, 64:]


def _project(xt, wcat):
    bn = xt.shape[0]
    rb = 2048
    grid = (bn // rb,)
    return pl.pallas_call(
        _proj_body,
        grid=grid,
        in_specs=[
            pl.BlockSpec((rb, 64), lambda i: (i, 0)),
            pl.BlockSpec((64, 128), lambda i: (0, 0)),
        ],
        out_specs=[
            pl.BlockSpec((rb, 128), lambda i: (i, 0)),
            pl.BlockSpec((rb, 64), lambda i: (i, 0)),
        ],
        out_shape=[
            jax.ShapeDtypeStruct((bn, 128), jnp.float32),
            jax.ShapeDtypeStruct((bn, 64), jnp.float32),
        ],
    )(xt, wcat)


def _sc_gather_planes(pt128, idx_t):
    kk, bn = idx_t.shape
    rows_per_w = bn // _NW
    ch = 128
    nch = rows_per_w // ch
    mesh = plsc.VectorSubcoreMesh(core_axis_name="c", subcore_axis_name="s")

    @functools.partial(
        pl.kernel,
        mesh=mesh,
        out_type=jax.ShapeDtypeStruct((kk, bn, 128), jnp.float32),
        scratch_types=[
            pltpu.VMEM((ch,), jnp.int32),
            pltpu.VMEM((ch, 128), jnp.float32),
            pltpu.SemaphoreType.DMA,
        ],
    )
    def k(pt_hbm, idxt_hbm, g_hbm, idx_v, rows_v, sem):
        wid = lax.axis_index("c") * _NS + lax.axis_index("s")
        base = wid * rows_per_w

        def step(i, carry):
            kplane = i // nch
            off = base + (i % nch) * ch
            pltpu.sync_copy(idxt_hbm.at[kplane, pl.ds(off, ch)], idx_v)
            pltpu.async_copy(pt_hbm.at[idx_v], rows_v, sem).wait()
            pltpu.sync_copy(rows_v, g_hbm.at[kplane, pl.ds(off, ch)])
            return carry

        lax.fori_loop(0, kk * nch, step, 0)

    return k(pt128, idx_t)


def _stats1_body(g_ref, qt_ref, s_ref, s2_ref, *, kk):
    q = qt_ref[...]
    s = jnp.zeros((64,), jnp.float32)
    s2 = jnp.zeros((64,), jnp.float32)
    for k in range(kk):
        y = g_ref[k][:, :64] + q
        s = s + jnp.sum(y, axis=0)
        s2 = s2 + jnp.sum(y * y, axis=0)
    s_ref[...] = jnp.broadcast_to(s[None, None, :], (1, 8, 64))
    s2_ref[...] = jnp.broadcast_to(s2[None, None, :], (1, 8, 64))


def _stats1(g, qt):
    kk, bn, _ = g.shape
    rb = 512
    nb = bn // rb
    return pl.pallas_call(
        functools.partial(_stats1_body, kk=kk),
        grid=(nb,),
        in_specs=[
            pl.BlockSpec((kk, rb, 128), lambda i: (0, i, 0)),
            pl.BlockSpec((rb, 64), lambda i: (i, 0)),
        ],
        out_specs=[
            pl.BlockSpec((1, 8, 64), lambda i: (i, 0, 0)),
            pl.BlockSpec((1, 8, 64), lambda i: (i, 0, 0)),
        ],
        out_shape=[
            jax.ShapeDtypeStruct((nb, 8, 64), jnp.float32),
            jax.ShapeDtypeStruct((nb, 8, 64), jnp.float32),
        ],
    )(g, qt)


def _main_body(g_ref, qt_ref, w2t_ref, prm_ref, m_ref, a2_ref, sa_ref, *, kk):
    q = qt_ref[...]
    sc1 = prm_ref[0:1, :]
    tc1 = prm_ref[1:2, :]
    w2t = w2t_ref[...]
    rb = q.shape[0]
    sa = jnp.zeros((64,), jnp.float32)
    acc = jnp.zeros((64, 64), jnp.float32)
    m = jnp.full((rb, 64), -jnp.inf, jnp.float32)
    for k in range(kk):
        a = _lrelu((g_ref[k][:, :64] + q) * sc1 + tc1)
        sa = sa + jnp.sum(a, axis=0)
        acc = acc + lax.dot_general(
            a, a, (((0,), (0,)), ((), ())), preferred_element_type=jnp.float32
        )
        y2 = jnp.dot(a, w2t, preferred_element_type=jnp.float32)
        m = jnp.maximum(m, y2)
    m_ref[...] = m
    a2_ref[...] = acc[None]
    sa_ref[...] = jnp.broadcast_to(sa[None, None, :], (1, 8, 64))


def _main(g, qt, w2t, prm):
    kk, bn, _ = g.shape
    rb = 512
    nb = bn // rb
    return pl.pallas_call(
        functools.partial(_main_body, kk=kk),
        grid=(nb,),
        in_specs=[
            pl.BlockSpec((kk, rb, 128), lambda i: (0, i, 0)),
            pl.BlockSpec((rb, 64), lambda i: (i, 0)),
            pl.BlockSpec((64, 64), lambda i: (0, 0)),
            pl.BlockSpec((8, 64), lambda i: (0, 0)),
        ],
        out_specs=[
            pl.BlockSpec((rb, 64), lambda i: (i, 0)),
            pl.BlockSpec((1, 64, 64), lambda i: (i, 0, 0)),
            pl.BlockSpec((1, 8, 64), lambda i: (i, 0, 0)),
        ],
        out_shape=[
            jax.ShapeDtypeStruct((bn, 64), jnp.float32),
            jax.ShapeDtypeStruct((nb, 64, 64), jnp.float32),
            jax.ShapeDtypeStruct((nb, 8, 64), jnp.float32),
        ],
    )(g, qt, w2t, prm)


def _epi_body(m_ref, prm_ref, o_ref):
    o_ref[...] = _lrelu(m_ref[...] * prm_ref[0:1, :] + prm_ref[1:2, :])


def _epilogue(m, prm2):
    bn = m.shape[0]
    rb = 2048
    return pl.pallas_call(
        _epi_body,
        grid=(bn // rb,),
        in_specs=[
            pl.BlockSpec((rb, 64), lambda i: (i, 0)),
            pl.BlockSpec((8, 64), lambda i: (0, 0)),
        ],
        out_specs=pl.BlockSpec((rb, 64), lambda i: (i, 0)),
        out_shape=jax.ShapeDtypeStruct((bn, 64), jnp.float32),
    )(m, prm2)


def _pack_prm(sc, tc):
    return jnp.concatenate(
        [sc[None, :], tc[None, :], jnp.zeros((6, 64), jnp.float32)], axis=0
    )


@jax.jit
def _impl(x, idx, W1, g1, b1, W2, g2, b2):
    B, F, N = x.shape
    K = idx.shape[-1]
    bn = B * N
    cnt = jnp.float32(B * N * K)

    xt = jnp.transpose(x, (0, 2, 1)).reshape(bn, F)
    w1a = W1[:, :F]
    w1b = W1[:, F:]
    wcat = jnp.concatenate([w1a.T, (w1b - w1a).T], axis=1)

    idx32 = idx.astype(jnp.int32) + (jnp.arange(B, dtype=jnp.int32) * N)[:, None, None]
    idx_t = jnp.transpose(idx32.reshape(bn, K), (1, 0))

    pt, qt = _project(xt, wcat)
    g = _sc_gather_planes(pt, idx_t)

    s_p, s2_p = _stats1(g, qt)
    s1 = jnp.sum(s_p[:, 0, :], axis=0)
    s2 = jnp.sum(s2_p[:, 0, :], axis=0)
    m1 = s1 / cnt
    v1 = s2 / cnt - m1 * m1
    sc1 = g1 / jnp.sqrt(v1 + _EPS)
    tc1 = b1 - m1 * sc1

    m, a2_p, sa_p = _main(g, qt, W2.T, _pack_prm(sc1, tc1))
    sa = jnp.sum(sa_p[:, 0, :], axis=0)
    amat = jnp.sum(a2_p, axis=0)
    m2 = (W2 @ sa) / cnt
    ey2sq = jnp.sum((W2 @ amat) * W2, axis=1) / cnt
    v2 = ey2sq - m2 * m2
    sc2 = g2 / jnp.sqrt(v2 + _EPS)
    tc2 = b2 - m2 * sc2

    out = _epilogue(m, _pack_prm(sc2, tc2))
    return jnp.transpose(out.reshape(B, N, 64), (0, 2, 1))


def kernel(x, fixed_knn_graph, W1, g1, b1, W2, g2, b2):
    return _impl(x, fixed_knn_graph, W1, g1, b1, W2, g2, b2)


# speedup vs baseline: 18.8921x; 1.2694x over previous
import functools

import jax
import jax.numpy as jnp
from jax import lax
from jax.experimental import pallas as pl
from jax.experimental.pallas import tpu as pltpu
from jax.experimental.pallas import tpu_sc as plsc

_NEG_SLOPE = 0.2
_EPS = 1e-5

_NC = 2
_NS = 16
_NW = _NC * _NS


def _lrelu(v):
    return jnp.where(v >= 0, v, _NEG_SLOPE * v)


def _proj_body(xt_ref, w_ref, pt_ref, qt_ref):
    y = jnp.dot(xt_ref[...], w_ref[...], preferred_element_type=jnp.float32)
    pt_ref[...] = y[:, :64]
    qt_ref[...] = y[:, 64:]


def _project(xt, wcat):
    bn = xt.shape[0]
    rb = 2048
    grid = (bn // rb,)
    return pl.pallas_call(
        _proj_body,
        grid=grid,
        in_specs=[
            pl.BlockSpec((rb, 64), lambda i: (i, 0)),
            pl.BlockSpec((64, 128), lambda i: (0, 0)),
        ],
        out_specs=[
            pl.BlockSpec((rb, 64), lambda i: (i, 0)),
            pl.BlockSpec((rb, 64), lambda i: (i, 0)),
        ],
        out_shape=[
            jax.ShapeDtypeStruct((bn, 64), jnp.float32),
            jax.ShapeDtypeStruct((bn, 64), jnp.float32),
        ],
    )(xt, wcat)


def _sc_gather_planes(pt, idx_t):
    kk, bn = idx_t.shape
    rows_per_w = bn // _NW
    ch = 128
    nch = rows_per_w // ch
    mesh = plsc.VectorSubcoreMesh(core_axis_name="c", subcore_axis_name="s")

    @functools.partial(
        pl.kernel,
        mesh=mesh,
        out_type=jax.ShapeDtypeStruct((kk, bn, 64), jnp.float32),
        scratch_types=[
            pltpu.VMEM((ch,), jnp.int32),
            pltpu.VMEM((ch, 64), jnp.float32),
            pltpu.SemaphoreType.DMA,
        ],
        compiler_params=pltpu.CompilerParams(use_tc_tiling_on_sc=False),
    )
    def k(pt_hbm, idxt_hbm, g_hbm, idx_v, rows_v, sem):
        wid = lax.axis_index("c") * _NS + lax.axis_index("s")
        base = wid * rows_per_w

        def step(i, carry):
            kplane = i // nch
            off = base + (i % nch) * ch
            pltpu.sync_copy(idxt_hbm.at[kplane, pl.ds(off, ch)], idx_v)
            pltpu.async_copy(pt_hbm.at[idx_v], rows_v, sem).wait()
            pltpu.sync_copy(rows_v, g_hbm.at[kplane, pl.ds(off, ch)])
            return carry

        lax.fori_loop(0, kk * nch, step, 0)

    return k(pt, idx_t)


def _stats1_body(g_ref, qp_ref, s_ref, s2_ref, *, kk):
    q = qp_ref[...]
    s = jnp.zeros((128,), jnp.float32)
    s2 = jnp.zeros((128,), jnp.float32)
    for k in range(kk):
        y = g_ref[k] + q
        s = s + jnp.sum(y, axis=0)
        s2 = s2 + jnp.sum(y * y, axis=0)
    s_ref[...] = jnp.broadcast_to(s[None, None, :], (1, 8, 128))
    s2_ref[...] = jnp.broadcast_to(s2[None, None, :], (1, 8, 128))


def _stats1(gp, qp):
    kk, bn2, _ = gp.shape
    rb = 256
    nb = bn2 // rb
    return pl.pallas_call(
        functools.partial(_stats1_body, kk=kk),
        grid=(nb,),
        in_specs=[
            pl.BlockSpec((kk, rb, 128), lambda i: (0, i, 0)),
            pl.BlockSpec((rb, 128), lambda i: (i, 0)),
        ],
        out_specs=[
            pl.BlockSpec((1, 8, 128), lambda i: (i, 0, 0)),
            pl.BlockSpec((1, 8, 128), lambda i: (i, 0, 0)),
        ],
        out_shape=[
            jax.ShapeDtypeStruct((nb, 8, 128), jnp.float32),
            jax.ShapeDtypeStruct((nb, 8, 128), jnp.float32),
        ],
    )(gp, qp)


def _main_body(g_ref, qp_ref, w2t_ref, prm_ref, m_ref, a2_ref, sa_ref, *, kk):
    q = qp_ref[...]
    sc1 = prm_ref[0:1, :]
    tc1 = prm_ref[1:2, :]
    w2t = w2t_ref[...]
    rb = q.shape[0]
    sa = jnp.zeros((128,), jnp.float32)
    acc = jnp.zeros((64, 64), jnp.float32)
    me = jnp.full((rb, 64), -jnp.inf, jnp.float32)
    mo = jnp.full((rb, 64), -jnp.inf, jnp.float32)
    cdims = (((0,), (0,)), ((), ()))
    for k in range(kk):
        a = _lrelu((g_ref[k] + q) * sc1 + tc1)
        ae = a[:, :64]
        ao = a[:, 64:]
        sa = sa + jnp.sum(a, axis=0)
        acc = acc + lax.dot_general(ae, ae, cdims, preferred_element_type=jnp.float32)
        acc = acc + lax.dot_general(ao, ao, cdims, preferred_element_type=jnp.float32)
        me = jnp.maximum(me, jnp.dot(ae, w2t, preferred_element_type=jnp.float32))
        mo = jnp.maximum(mo, jnp.dot(ao, w2t, preferred_element_type=jnp.float32))
    m_ref[...] = jnp.concatenate([me, mo], axis=1)
    a2_ref[...] = acc[None]
    sa_ref[...] = jnp.broadcast_to(sa[None, None, :], (1, 8, 128))


def _main(gp, qp, w2t, prm):
    kk, bn2, _ = gp.shape
    rb = 256
    nb = bn2 // rb
    return pl.pallas_call(
        functools.partial(_main_body, kk=kk),
        grid=(nb,),
        in_specs=[
            pl.BlockSpec((kk, rb, 128), lambda i: (0, i, 0)),
            pl.BlockSpec((rb, 128), lambda i: (i, 0)),
            pl.BlockSpec((64, 64), lambda i: (0, 0)),
            pl.BlockSpec((8, 128), lambda i: (0, 0)),
        ],
        out_specs=[
            pl.BlockSpec((rb, 128), lambda i: (i, 0)),
            pl.BlockSpec((1, 64, 64), lambda i: (i, 0, 0)),
            pl.BlockSpec((1, 8, 128), lambda i: (i, 0, 0)),
        ],
        out_shape=[
            jax.ShapeDtypeStruct((bn2, 128), jnp.float32),
            jax.ShapeDtypeStruct((nb, 64, 64), jnp.float32),
            jax.ShapeDtypeStruct((nb, 8, 128), jnp.float32),
        ],
    )(gp, qp, w2t, prm)


def _epi_body(m_ref, prm_ref, o_ref):
    o_ref[...] = _lrelu(m_ref[...] * prm_ref[0:1, :] + prm_ref[1:2, :])


def _epilogue(m, prm2):
    bn2 = m.shape[0]
    rb = 1024
    return pl.pallas_call(
        _epi_body,
        grid=(bn2 // rb,),
        in_specs=[
            pl.BlockSpec((rb, 128), lambda i: (i, 0)),
            pl.BlockSpec((8, 128), lambda i: (0, 0)),
        ],
        out_specs=pl.BlockSpec((rb, 128), lambda i: (i, 0)),
        out_shape=jax.ShapeDtypeStruct((bn2, 128), jnp.float32),
    )(m, prm2)


def _pack_prm(sc, tc):
    row0 = jnp.concatenate([sc, sc])[None, :]
    row1 = jnp.concatenate([tc, tc])[None, :]
    return jnp.concatenate([row0, row1, jnp.zeros((6, 128), jnp.float32)], axis=0)


@jax.jit
def _impl(x, idx, W1, g1, b1, W2, g2, b2):
    B, F, N = x.shape
    K = idx.shape[-1]
    bn = B * N
    cnt = jnp.float32(B * N * K)

    xt = jnp.transpose(x, (0, 2, 1)).reshape(bn, F)
    w1a = W1[:, :F]
    w1b = W1[:, F:]
    wcat = jnp.concatenate([w1a.T, (w1b - w1a).T], axis=1)

    idx32 = idx.astype(jnp.int32) + (jnp.arange(B, dtype=jnp.int32) * N)[:, None, None]
    idx_t = jnp.transpose(idx32.reshape(bn, K), (1, 0))

    pt, qt = _project(xt, wcat)
    g = _sc_gather_planes(pt, idx_t)
    gp = g.reshape(K, bn // 2, 128)
    qp = qt.reshape(bn // 2, 128)

    s_p, s2_p = _stats1(gp, qp)
    s1p = jnp.sum(s_p[:, 0, :], axis=0)
    s2p = jnp.sum(s2_p[:, 0, :], axis=0)
    s1 = s1p[:64] + s1p[64:]
    s2 = s2p[:64] + s2p[64:]
    m1 = s1 / cnt
    v1 = s2 / cnt - m1 * m1
    sc1 = g1 / jnp.sqrt(v1 + _EPS)
    tc1 = b1 - m1 * sc1

    m, a2_p, sa_p = _main(gp, qp, W2.T, _pack_prm(sc1, tc1))
    sap = jnp.sum(sa_p[:, 0, :], axis=0)
    sa = sap[:64] + sap[64:]
    amat = jnp.sum(a2_p, axis=0)
    m2 = (W2 @ sa) / cnt
    ey2sq = jnp.sum((W2 @ amat) * W2, axis=1) / cnt
    v2 = ey2sq - m2 * m2
    sc2 = g2 / jnp.sqrt(v2 + _EPS)
    tc2 = b2 - m2 * sc2

    out = _epilogue(m, _pack_prm(sc2, tc2))
    return jnp.transpose(out.reshape(B, N, 64), (0, 2, 1))


def kernel(x, fixed_knn_graph, W1, g1, b1, W2, g2, b2):
    return _impl(x, fixed_knn_graph, W1, g1, b1, W2, g2, b2)


# speedup vs baseline: 26.8983x; 1.4238x over previous
import functools

import jax
import jax.numpy as jnp
from jax import lax
from jax.experimental import pallas as pl
from jax.experimental.pallas import tpu as pltpu
from jax.experimental.pallas import tpu_sc as plsc

_NEG_SLOPE = 0.2
_EPS = 1e-5

_NC = 2
_NS = 16
_NW = _NC * _NS


def _lrelu(v):
    return jnp.where(v >= 0, v, _NEG_SLOPE * v)


def _proj_body(xt_ref, w_ref, pt_ref, qt_ref):
    y = jnp.dot(xt_ref[...], w_ref[...], preferred_element_type=jnp.float32)
    pt_ref[...] = y[:, :64]
    qt_ref[...] = y[:, 64:]


def _project(xt, wcat):
    bn = xt.shape[0]
    rb = 2048
    grid = (bn // rb,)
    return pl.pallas_call(
        _proj_body,
        grid=grid,
        in_specs=[
            pl.BlockSpec((rb, 64), lambda i: (i, 0)),
            pl.BlockSpec((64, 128), lambda i: (0, 0)),
        ],
        out_specs=[
            pl.BlockSpec((rb, 64), lambda i: (i, 0)),
            pl.BlockSpec((rb, 64), lambda i: (i, 0)),
        ],
        out_shape=[
            jax.ShapeDtypeStruct((bn, 64), jnp.float32),
            jax.ShapeDtypeStruct((bn, 64), jnp.float32),
        ],
    )(xt, wcat)


def _sc_gather_stats(pt, idx_w, qt):
    nw, kk, rpw = idx_w.shape
    bn = pt.shape[0]
    ch = 128
    nch = rpw // ch
    npairs = (kk * nch) // 2
    mesh = plsc.VectorSubcoreMesh(core_axis_name="c", subcore_axis_name="s")

    @functools.partial(
        pl.kernel,
        mesh=mesh,
        out_type=(
            jax.ShapeDtypeStruct((kk, bn, 64), jnp.float32),
            jax.ShapeDtypeStruct((nw, 8, 16), jnp.float32),
        ),
        scratch_types=[
            pltpu.VMEM((kk, rpw), jnp.int32),
            pltpu.VMEM((rpw, 64), jnp.float32),
            pltpu.VMEM((ch, 64), jnp.float32),
            pltpu.VMEM((ch, 64), jnp.float32),
            pltpu.VMEM((8, 16), jnp.float32),
            pltpu.SemaphoreType.DMA,
            pltpu.SemaphoreType.DMA,
            pltpu.SemaphoreType.DMA,
            pltpu.SemaphoreType.DMA,
        ],
        compiler_params=pltpu.CompilerParams(use_tc_tiling_on_sc=False),
    )
    def k(pt_hbm, idxw_hbm, q_hbm, g_hbm, st_hbm, idx_v, q_v, buf0, buf1,
          st_v, gs0, gs1, ws0, ws1):
        wid = lax.axis_index("c") * _NS + lax.axis_index("s")
        base = wid * rpw
        pltpu.sync_copy(idxw_hbm.at[wid], idx_v)
        pltpu.sync_copy(q_hbm.at[pl.ds(base, rpw)], q_v)

        def start_gather(i, buf, sem):
            kplane = i // nch
            roff = lax.rem(i, nch) * ch
            pltpu.async_copy(
                pt_hbm.at[idx_v.at[kplane, pl.ds(roff, ch)]], buf, sem
            )

        def start_write(i, buf, sem):
            kplane = i // nch
            off = base + lax.rem(i, nch) * ch
            pltpu.async_copy(buf, g_hbm.at[kplane, pl.ds(off, ch)], sem)

        def drain_gather(buf, sem):
            pltpu.make_async_copy(pt_hbm.at[pl.ds(0, ch)], buf, sem).wait()

        def drain_write(buf, sem):
            pltpu.make_async_copy(buf, g_hbm.at[0, pl.ds(0, ch)], sem).wait()

        def stats_chunk(i, buf, carry):
            qoff = lax.rem(i, nch) * ch

            def row(r, c):
                out = []
                for j in range(4):
                    pg = buf[r, pl.ds(16 * j, 16)]
                    qv = q_v[qoff + r, pl.ds(16 * j, 16)]
                    y = pg + qv
                    out.append(c[j] + y)
                    out.append(c[4 + j] + y * y)
                return (out[0], out[2], out[4], out[6],
                        out[1], out[3], out[5], out[7])

            return lax.fori_loop(0, ch, row, carry)

        start_gather(0, buf0, gs0)

        def pair(t, carry):
            i0 = 2 * t

            @pl.when(t > 0)
            def _():
                drain_write(buf1, ws1)

            start_gather(i0 + 1, buf1, gs1)
            drain_gather(buf0, gs0)
            carry = stats_chunk(i0, buf0, carry)
            start_write(i0, buf0, ws0)
            drain_write(buf0, ws0)

            @pl.when(t + 1 < npairs)
            def _():
                start_gather(i0 + 2, buf0, gs0)

            drain_gather(buf1, gs1)
            carry = stats_chunk(i0 + 1, buf1, carry)
            start_write(i0 + 1, buf1, ws1)
            return carry

        zero = jnp.zeros((16,), jnp.float32)
        carry = lax.fori_loop(0, npairs, pair, (zero,) * 8)
        for j in range(8):
            st_v[j] = carry[j]
        pltpu.sync_copy(st_v, st_hbm.at[wid])
        drain_write(buf1, ws1)

    return k(pt, idx_w, qt)


def _main_body(g_ref, qp_ref, w2t_ref, prm_ref, m_ref, a2_ref, sa_ref, *, kk):
    q = qp_ref[...]
    sc1 = prm_ref[0:1, :]
    tc1 = prm_ref[1:2, :]
    w2t = w2t_ref[...]
    rb = q.shape[0]
    sa = jnp.zeros((128,), jnp.float32)
    acc = jnp.zeros((64, 64), jnp.float32)
    me = jnp.full((rb, 64), -jnp.inf, jnp.float32)
    mo = jnp.full((rb, 64), -jnp.inf, jnp.float32)
    cdims = (((0,), (0,)), ((), ()))
    for k in range(kk):
        a = _lrelu((g_ref[k] + q) * sc1 + tc1)
        ae = a[:, :64]
        ao = a[:, 64:]
        sa = sa + jnp.sum(a, axis=0)
        acc = acc + lax.dot_general(ae, ae, cdims, preferred_element_type=jnp.float32)
        acc = acc + lax.dot_general(ao, ao, cdims, preferred_element_type=jnp.float32)
        me = jnp.maximum(me, jnp.dot(ae, w2t, preferred_element_type=jnp.float32))
        mo = jnp.maximum(mo, jnp.dot(ao, w2t, preferred_element_type=jnp.float32))
    m_ref[...] = jnp.concatenate([me, mo], axis=1)
    a2_ref[...] = acc[None]
    sa_ref[...] = jnp.broadcast_to(sa[None, None, :], (1, 8, 128))


def _main(gp, qp, w2t, prm):
    kk, bn2, _ = gp.shape
    rb = 256
    nb = bn2 // rb
    return pl.pallas_call(
        functools.partial(_main_body, kk=kk),
        grid=(nb,),
        in_specs=[
            pl.BlockSpec((kk, rb, 128), lambda i: (0, i, 0)),
            pl.BlockSpec((rb, 128), lambda i: (i, 0)),
            pl.BlockSpec((64, 64), lambda i: (0, 0)),
            pl.BlockSpec((8, 128), lambda i: (0, 0)),
        ],
        out_specs=[
            pl.BlockSpec((rb, 128), lambda i: (i, 0)),
            pl.BlockSpec((1, 64, 64), lambda i: (i, 0, 0)),
            pl.BlockSpec((1, 8, 128), lambda i: (i, 0, 0)),
        ],
        out_shape=[
            jax.ShapeDtypeStruct((bn2, 128), jnp.float32),
            jax.ShapeDtypeStruct((nb, 64, 64), jnp.float32),
            jax.ShapeDtypeStruct((nb, 8, 128), jnp.float32),
        ],
    )(gp, qp, w2t, prm)


def _epi_body(m_ref, prm_ref, o_ref):
    o_ref[...] = _lrelu(m_ref[...] * prm_ref[0:1, :] + prm_ref[1:2, :])


def _epilogue(m, prm2):
    bn2 = m.shape[0]
    rb = 1024
    return pl.pallas_call(
        _epi_body,
        grid=(bn2 // rb,),
        in_specs=[
            pl.BlockSpec((rb, 128), lambda i: (i, 0)),
            pl.BlockSpec((8, 128), lambda i: (0, 0)),
        ],
        out_specs=pl.BlockSpec((rb, 128), lambda i: (i, 0)),
        out_shape=jax.ShapeDtypeStruct((bn2, 128), jnp.float32),
    )(m, prm2)


def _pack_prm(sc, tc):
    row0 = jnp.concatenate([sc, sc])[None, :]
    row1 = jnp.concatenate([tc, tc])[None, :]
    return jnp.concatenate([row0, row1, jnp.zeros((6, 128), jnp.float32)], axis=0)


@jax.jit
def _impl(x, idx, W1, g1, b1, W2, g2, b2):
    B, F, N = x.shape
    K = idx.shape[-1]
    bn = B * N
    cnt = jnp.float32(B * N * K)

    xt = jnp.transpose(x, (0, 2, 1)).reshape(bn, F)
    w1a = W1[:, :F]
    w1b = W1[:, F:]
    wcat = jnp.concatenate([w1a.T, (w1b - w1a).T], axis=1)

    idx32 = idx.astype(jnp.int32) + (jnp.arange(B, dtype=jnp.int32) * N)[:, None, None]
    idx_t = jnp.transpose(idx32.reshape(bn, K), (1, 0))
    rpw = bn // _NW
    idx_w = jnp.transpose(idx_t.reshape(K, _NW, rpw), (1, 0, 2))

    pt, qt = _project(xt, wcat)
    g, st = _sc_gather_stats(pt, idx_w, qt)
    gp = g.reshape(K, bn // 2, 128)
    qp = qt.reshape(bn // 2, 128)

    s1 = jnp.sum(st[:, 0:4, :].reshape(_NW, 64), axis=0)
    s2 = jnp.sum(st[:, 4:8, :].reshape(_NW, 64), axis=0)
    m1 = s1 / cnt
    v1 = s2 / cnt - m1 * m1
    sc1 = g1 / jnp.sqrt(v1 + _EPS)
    tc1 = b1 - m1 * sc1

    m, a2_p, sa_p = _main(gp, qp, W2.T, _pack_prm(sc1, tc1))
    sap = jnp.sum(sa_p[:, 0, :], axis=0)
    sa = sap[:64] + sap[64:]
    amat = jnp.sum(a2_p, axis=0)
    m2 = (W2 @ sa) / cnt
    ey2sq = jnp.sum((W2 @ amat) * W2, axis=1) / cnt
    v2 = ey2sq - m2 * m2
    sc2 = g2 / jnp.sqrt(v2 + _EPS)
    tc2 = b2 - m2 * sc2

    out = _epilogue(m, _pack_prm(sc2, tc2))
    return jnp.transpose(out.reshape(B, N, 64), (0, 2, 1))


def kernel(x, fixed_knn_graph, W1, g1, b1, W2, g2, b2):
    return _impl(x, fixed_knn_graph, W1, g1, b1, W2, g2, b2)
